# baseline (speedup 1.0000x reference)
import functools

import jax
import jax.numpy as jnp
from jax import lax
from jax.experimental import pallas as pl
from jax.experimental.pallas import tpu as pltpu
from jax.experimental.pallas import tpu_sc as plsc

N = 50000
E = 800000
NW = 32
NS = 16
BATCH = 128
KPT = 200
EPAD = NW * KPT * BATCH
BR = 1568
NPAD = 32 * BR
SLICE = NPAD // NS
GRP = 4
EPS = 1e-5


def _sc_aggregate(nc, wc):
    mesh = plsc.VectorSubcoreMesh(core_axis_name="c", subcore_axis_name="s")
    out_t = jax.ShapeDtypeStruct((2 * nc * NPAD, wc), jnp.float32)
    scratch = [
        pltpu.VMEM((KPT, BATCH), jnp.int32),
        pltpu.VMEM((KPT, BATCH), jnp.int32),
        pltpu.VMEM((3 * GRP, BATCH, wc), jnp.float32),
        pltpu.VMEM_SHARED((NPAD, wc), jnp.float32),
        pltpu.SemaphoreType.DMA,
        pltpu.SemaphoreType.DMA,
    ]

    @functools.partial(
        pl.kernel, out_type=out_t, mesh=mesh, scratch_types=scratch,
        compiler_params=pltpu.CompilerParams(use_tc_tiling_on_sc=False))
    def k(*refs):
        x_hbm = refs[:nc]
        src_hbm, dst_hbm, zero_hbm, out_hbm = refs[nc:nc + 4]
        src_v, dst_v, rows_v, acc, gsem, ssem = refs[nc + 4:]
        c = lax.axis_index("c")
        s = lax.axis_index("s")
        wid = c * NS + s
        pltpu.sync_copy(src_hbm.at[pl.ds(wid * KPT, KPT)], src_v)
        pltpu.sync_copy(dst_hbm.at[pl.ds(wid * KPT, KPT)], dst_v)
        for cc in range(nc):
            pltpu.sync_copy(zero_hbm, acc.at[pl.ds(s * SLICE, SLICE)])
            plsc.subcore_barrier()

            ngrp = KPT // GRP
            for k in range(GRP):
                pltpu.async_copy(x_hbm[cc].at[src_v.at[k]],
                                 rows_v.at[k], gsem)

            def body(g, carry):
                slot = lax.rem(g, 3) * GRP
                nslot = lax.rem(g + 1, 3) * GRP
                pslot = lax.rem(g + 2, 3) * GRP
                base = g * GRP
                for k in range(GRP):
                    pltpu.make_async_copy(x_hbm[cc].at[src_v.at[base + k]],
                                          rows_v.at[slot + k], gsem).wait()

                @pl.when(g >= 1)
                def _():
                    for k in range(GRP):
                        pltpu.make_async_copy(
                            rows_v.at[pslot + k],
                            acc.at[dst_v.at[base - GRP + k]], ssem).wait()

                @pl.when(g < ngrp - 1)
                def _():
                    for k in range(GRP):
                        pltpu.async_copy(
                            x_hbm[cc].at[src_v.at[base + GRP + k]],
                            rows_v.at[nslot + k], gsem)

                for k in range(GRP):
                    pltpu.async_copy(rows_v.at[slot + k],
                                     acc.at[dst_v.at[base + k]], ssem,
                                     add=True)
                return carry

            lax.fori_loop(0, ngrp, body, 0)
            lslot = ((ngrp - 1) % 3) * GRP
            for k in range(GRP):
                pltpu.make_async_copy(
                    rows_v.at[lslot + k],
                    acc.at[dst_v.at[(ngrp - 1) * GRP + k]], ssem).wait()
            plsc.subcore_barrier()
            pltpu.sync_copy(
                acc.at[pl.ds(s * SLICE, SLICE)],
                out_hbm.at[pl.ds((c * nc + cc) * NPAD + s * SLICE, SLICE)])
            plsc.subcore_barrier()

    return k


def _row_mask(i):
    rows = lax.broadcasted_iota(jnp.int32, (BR, 1), 0) + i * BR
    return rows < N


def _pass_a(S, xfull, dinv, W, b, din, dout):
    def body(s_ref, x_ref, dinv_ref, w_ref, b_ref, h_ref, st_ref):
        i = pl.program_id(0)
        u = dinv_ref[...] * (s_ref[0] + s_ref[1] + x_ref[...])
        part = b_ref[...] + lax.dot_general(
            u, w_ref[...], (((1,), (1,)), ((), ())))
        h_ref[...] = part
        m = _row_mask(i)
        hm = jnp.where(m, part, 0.0)
        s1 = jnp.sum(hm, axis=0, keepdims=True)
        s2 = jnp.sum(hm * part, axis=0, keepdims=True)
        blk = jnp.concatenate(
            [s1, s2, jnp.zeros((6, dout), jnp.float32)], axis=0)

        @pl.when(i == 0)
        def _():
            st_ref[...] = blk

        @pl.when(i > 0)
        def _():
            st_ref[...] += blk

    return pl.pallas_call(
        body,
        grid=(NPAD // BR,),
        in_specs=[
            pl.BlockSpec((2, BR, din), lambda i: (0, i, 0)),
            pl.BlockSpec((BR, din), lambda i: (i, 0)),
            pl.BlockSpec((BR, 1), lambda i: (i, 0)),
            pl.BlockSpec((dout, din), lambda i: (0, 0)),
            pl.BlockSpec((1, dout), lambda i: (0, 0)),
        ],
        out_specs=[
            pl.BlockSpec((BR, dout), lambda i: (i, 0)),
            pl.BlockSpec((8, dout), lambda i: (0, 0)),
        ],
        out_shape=[
            jax.ShapeDtypeStruct((NPAD, dout), jnp.float32),
            jax.ShapeDtypeStruct((8, dout), jnp.float32),
        ],
    )(S, xfull, dinv, W, b)


def _bn_leaky(h, st_ref, g_ref, be_ref):
    mean = st_ref[0:1, :] / N
    var = st_ref[1:2, :] / N - mean * mean
    y = (h - mean) * lax.rsqrt(var + EPS) * g_ref[...] + be_ref[...]
    return jnp.where(y >= 0, y, 0.1 * y)


def _pass_b(h, st, dinv, g, be, dout, nco, wco):
    def body(h_ref, st_ref, dinv_ref, g_ref, be_ref, full_ref, *o_refs):
        z = dinv_ref[...] * _bn_leaky(h_ref[...], st_ref, g_ref, be_ref)
        full_ref[...] = z
        for c in range(nco):
            o_refs[c][...] = z[:, c * wco:(c + 1) * wco]

    return pl.pallas_call(
        body,
        grid=(NPAD // BR,),
        in_specs=[
            pl.BlockSpec((BR, dout), lambda i: (i, 0)),
            pl.BlockSpec((8, dout), lambda i: (0, 0)),
            pl.BlockSpec((BR, 1), lambda i: (i, 0)),
            pl.BlockSpec((1, dout), lambda i: (0, 0)),
            pl.BlockSpec((1, dout), lambda i: (0, 0)),
        ],
        out_specs=[pl.BlockSpec((BR, dout), lambda i: (i, 0))]
        + [pl.BlockSpec((BR, wco), lambda i: (i, 0)) for _ in range(nco)],
        out_shape=[jax.ShapeDtypeStruct((NPAD, dout), jnp.float32)]
        + [jax.ShapeDtypeStruct((NPAD, wco), jnp.float32)
           for _ in range(nco)],
    )(h, st, dinv, g, be)


def _pass_b5(h, st, dinv, g, be, W6):
    def body(h_ref, st_ref, dinv_ref, g_ref, be_ref, w6_ref, o_ref):
        y = _bn_leaky(h_ref[...], st_ref, g_ref, be_ref)
        t = lax.dot_general(y, w6_ref[...], (((1,), (1,)), ((), ())))
        t8 = jnp.pad(t, ((0, 0), (0, 7)))
        o_ref[...] = dinv_ref[...] * t8

    return pl.pallas_call(
        body,
        grid=(NPAD // BR,),
        in_specs=[
            pl.BlockSpec((BR, 512), lambda i: (i, 0)),
            pl.BlockSpec((8, 512), lambda i: (0, 0)),
            pl.BlockSpec((BR, 1), lambda i: (i, 0)),
            pl.BlockSpec((1, 512), lambda i: (0, 0)),
            pl.BlockSpec((1, 512), lambda i: (0, 0)),
            pl.BlockSpec((1, 512), lambda i: (0, 0)),
        ],
        out_specs=pl.BlockSpec((BR, 8), lambda i: (i, 0)),
        out_shape=jax.ShapeDtypeStruct((NPAD, 8), jnp.float32),
    )(h, st, dinv, g, be, W6)


def _prologue(x8, degS):
    def body(x_ref, d_ref, dinv_ref, o_ref):
        i = pl.program_id(0)
        deg = d_ref[0, 0, :, 0:1] + d_ref[1, 0, :, 0:1] + 1.0
        dinv = jnp.where(_row_mask(i), lax.rsqrt(deg), 0.0)
        dinv_ref[...] = dinv
        o_ref[...] = dinv * x_ref[...]

    return pl.pallas_call(
        body,
        grid=(NPAD // BR,),
        in_specs=[
            pl.BlockSpec((BR, 8), lambda i: (i, 0)),
            pl.BlockSpec((2, 1, BR, 8), lambda i: (0, 0, i, 0)),
        ],
        out_specs=[
            pl.BlockSpec((BR, 1), lambda i: (i, 0)),
            pl.BlockSpec((BR, 8), lambda i: (i, 0)),
        ],
        out_shape=[
            jax.ShapeDtypeStruct((NPAD, 1), jnp.float32),
            jax.ShapeDtypeStruct((NPAD, 8), jnp.float32),
        ],
    )(x8, degS)


def _final(S6, h6c, dinv, b6):
    def body(s_ref, h_ref, dinv_ref, b_ref, o_ref):
        v = dinv_ref[...] * (s_ref[0, 0, :, 0:1] + s_ref[1, 0, :, 0:1]
                             + h_ref[:, 0:1]) + b_ref[...]
        o_ref[...] = jax.nn.sigmoid(v)

    return pl.pallas_call(
        body,
        grid=(NPAD // BR,),
        in_specs=[
            pl.BlockSpec((2, 1, BR, 8), lambda i: (0, 0, i, 0)),
            pl.BlockSpec((BR, 8), lambda i: (i, 0)),
            pl.BlockSpec((BR, 1), lambda i: (i, 0)),
            pl.BlockSpec((1, 1), lambda i: (0, 0)),
        ],
        out_specs=pl.BlockSpec((BR, 1), lambda i: (i, 0)),
        out_shape=jax.ShapeDtypeStruct((N, 1), jnp.float32),
    )(S6, h6c, dinv, b6)


def _repack(Sflat, nc, wc):
    S = Sflat.reshape(2, nc, NPAD, wc)
    if nc == 1:
        return S.reshape(2, NPAD, wc), S
    return S.transpose(0, 2, 1, 3).reshape(2, NPAD, nc * wc), S


def kernel(x, edge_index, W1, b1, W2, b2, W3, b3, W4, b4, W5, b5, W6, b6,
           g1, be1, g2, be2, g3, be3, g4, be4, g5, be5):
    ei = edge_index.astype(jnp.int32)
    pad = jnp.full((EPAD - E,), N, jnp.int32)
    src3 = jnp.concatenate([ei[0], pad]).reshape(NW * KPT, BATCH)
    dst3 = jnp.concatenate([ei[1], pad]).reshape(NW * KPT, BATCH)

    x8 = jnp.pad(x, ((0, NPAD - N), (0, 5)))
    ones8 = jnp.ones((NPAD, 8), jnp.float32)
    z8 = jnp.zeros((SLICE, 8), jnp.float32)
    z16 = jnp.zeros((SLICE, 16), jnp.float32)
    W1p = jnp.pad(W1, ((0, 0), (0, 5)))

    agg8 = _sc_aggregate(1, 8)
    agg4 = _sc_aggregate(4, 16)
    agg16 = _sc_aggregate(16, 16)

    degF, degS = _repack(agg8(ones8, src3, dst3, z8), 1, 8)
    dinv, xc0 = _prologue(x8, degS)

    S0f, _ = _repack(agg8(xc0, src3, dst3, z8), 1, 8)
    h1, st1 = _pass_a(S0f, xc0, dinv, W1p, b1[None], 8, 64)
    x1, *xc1 = _pass_b(h1, st1, dinv, g1[None], be1[None], 64, 4, 16)

    S1f, _ = _repack(agg4(*xc1, src3, dst3, z16), 4, 16)
    h2, st2 = _pass_a(S1f, x1, dinv, W2, b2[None], 64, 64)
    x2, *xc2 = _pass_b(h2, st2, dinv, g2[None], be2[None], 64, 4, 16)

    S2f, _ = _repack(agg4(*xc2, src3, dst3, z16), 4, 16)
    h3, st3 = _pass_a(S2f, x2, dinv, W3, b3[None], 64, 256)
    x3, *xc3 = _pass_b(h3, st3, dinv, g3[None], be3[None], 256, 16, 16)

    S3f, _ = _repack(agg16(*xc3, src3, dst3, z16), 16, 16)
    h4, st4 = _pass_a(S3f, x3, dinv, W4, b4[None], 256, 256)
    x4, *xc4 = _pass_b(h4, st4, dinv, g4[None], be4[None], 256, 16, 16)

    S4f, _ = _repack(agg16(*xc4, src3, dst3, z16), 16, 16)
    h5, st5 = _pass_a(S4f, x4, dinv, W5, b5[None], 256, 512)
    h6c = _pass_b5(h5, st5, dinv, g5[None], be5[None], W6)

    _, S6 = _repack(agg8(h6c, src3, dst3, z8), 1, 8)
    return _final(S6, h6c, dinv, b6[None])


# speedup vs baseline: 1.4819x; 1.4819x over previous
import functools

import jax
import jax.numpy as jnp
from jax import lax
from jax.experimental import pallas as pl
from jax.experimental.pallas import tpu as pltpu
from jax.experimental.pallas import tpu_sc as plsc

N = 50000
E = 800000
NW = 32
NS = 16
BATCH = 128
KPT = 196
EPAD = NW * KPT * BATCH
BR = 1568
NPAD = 32 * BR
SLICE = NPAD // NS
GRP = 4
EPS = 1e-5


def _sc_aggregate(nc, wc):
    mesh = plsc.VectorSubcoreMesh(core_axis_name="c", subcore_axis_name="s")
    out_t = jax.ShapeDtypeStruct((2 * nc * NPAD, wc), jnp.float32)
    scratch = [
        pltpu.VMEM((KPT, BATCH), jnp.int32),
        pltpu.VMEM((KPT, BATCH), jnp.int32),
        pltpu.VMEM((3 * GRP, BATCH, wc), jnp.float32),
        pltpu.VMEM_SHARED((NPAD, wc), jnp.float32),
        pltpu.SemaphoreType.DMA,
        pltpu.SemaphoreType.DMA,
    ]

    @functools.partial(
        pl.kernel, out_type=out_t, mesh=mesh, scratch_types=scratch,
        compiler_params=pltpu.CompilerParams(use_tc_tiling_on_sc=False))
    def k(*refs):
        x_hbm = refs[:nc]
        src_hbm, dst_hbm, zero_hbm, out_hbm = refs[nc:nc + 4]
        src_v, dst_v, rows_v, acc, gsem, ssem = refs[nc + 4:]
        c = lax.axis_index("c")
        s = lax.axis_index("s")
        wid = c * NS + s
        pltpu.sync_copy(src_hbm.at[pl.ds(wid * KPT, KPT)], src_v)
        pltpu.sync_copy(dst_hbm.at[pl.ds(wid * KPT, KPT)], dst_v)
        for cc in range(nc):
            pltpu.sync_copy(zero_hbm, acc.at[pl.ds(s * SLICE, SLICE)])
            plsc.subcore_barrier()

            ngrp = KPT // GRP
            for k in range(GRP):
                pltpu.async_copy(x_hbm[cc].at[src_v.at[k]],
                                 rows_v.at[k], gsem)

            def body(g, carry):
                slot = lax.rem(g, 3) * GRP
                nslot = lax.rem(g + 1, 3) * GRP
                pslot = lax.rem(g + 2, 3) * GRP
                base = g * GRP
                for k in range(GRP):
                    pltpu.make_async_copy(x_hbm[cc].at[src_v.at[base + k]],
                                          rows_v.at[slot + k], gsem).wait()

                @pl.when(g >= 1)
                def _():
                    for k in range(GRP):
                        pltpu.make_async_copy(
                            rows_v.at[pslot + k],
                            acc.at[dst_v.at[base - GRP + k]], ssem).wait()

                @pl.when(g < ngrp - 1)
                def _():
                    for k in range(GRP):
                        pltpu.async_copy(
                            x_hbm[cc].at[src_v.at[base + GRP + k]],
                            rows_v.at[nslot + k], gsem)

                for k in range(GRP):
                    pltpu.async_copy(rows_v.at[slot + k],
                                     acc.at[dst_v.at[base + k]], ssem,
                                     add=True)
                return carry

            lax.fori_loop(0, ngrp, body, 0)
            lslot = ((ngrp - 1) % 3) * GRP
            for k in range(GRP):
                pltpu.make_async_copy(
                    rows_v.at[lslot + k],
                    acc.at[dst_v.at[(ngrp - 1) * GRP + k]], ssem).wait()
            plsc.subcore_barrier()
            pltpu.sync_copy(
                acc.at[pl.ds(s * SLICE, SLICE)],
                out_hbm.at[pl.ds((c * nc + cc) * NPAD + s * SLICE, SLICE)])
            plsc.subcore_barrier()

    return k


def _sc_count():
    mesh = plsc.VectorSubcoreMesh(core_axis_name="c", subcore_axis_name="s")
    out_t = jax.ShapeDtypeStruct((2 * NPAD, 8), jnp.float32)
    scratch = [
        pltpu.VMEM((KPT, BATCH), jnp.int32),
        pltpu.VMEM((BATCH, 8), jnp.float32),
        pltpu.VMEM_SHARED((NPAD, 8), jnp.float32),
        pltpu.SemaphoreType.DMA,
    ]

    @functools.partial(
        pl.kernel, out_type=out_t, mesh=mesh, scratch_types=scratch,
        compiler_params=pltpu.CompilerParams(use_tc_tiling_on_sc=False))
    def k(dst_hbm, ones_hbm, zero_hbm, out_hbm, dst_v, ones_v, acc, ssem):
        c = lax.axis_index("c")
        s = lax.axis_index("s")
        wid = c * NS + s
        pltpu.sync_copy(dst_hbm.at[pl.ds(wid * KPT, KPT)], dst_v)
        pltpu.sync_copy(ones_hbm, ones_v)
        pltpu.sync_copy(zero_hbm, acc.at[pl.ds(s * SLICE, SLICE)])
        plsc.subcore_barrier()

        def fire(j, carry):
            pltpu.async_copy(ones_v, acc.at[dst_v.at[j]], ssem, add=True)

            @pl.when(j >= GRP)
            def _():
                pltpu.make_async_copy(ones_v, acc.at[dst_v.at[j - GRP]],
                                      ssem).wait()

            return carry

        lax.fori_loop(0, KPT, fire, 0)
        for k2 in range(GRP):
            pltpu.make_async_copy(ones_v, acc.at[dst_v.at[KPT - GRP + k2]],
                                  ssem).wait()
        plsc.subcore_barrier()
        pltpu.sync_copy(acc.at[pl.ds(s * SLICE, SLICE)],
                        out_hbm.at[pl.ds(c * NPAD + s * SLICE, SLICE)])
        plsc.subcore_barrier()

    return k


def _row_mask(i):
    rows = lax.broadcasted_iota(jnp.int32, (BR, 1), 0) + i * BR
    return rows < N


def _pass_a(S, xfull, dinv, W, b, din, dout):
    def body(s_ref, x_ref, dinv_ref, w_ref, b_ref, h_ref, st_ref):
        i = pl.program_id(0)
        u = dinv_ref[...] * (s_ref[0] + s_ref[1] + x_ref[...])
        part = b_ref[...] + lax.dot_general(
            u, w_ref[...], (((1,), (1,)), ((), ())))
        h_ref[...] = part
        m = _row_mask(i)
        hm = jnp.where(m, part, 0.0)
        s1 = jnp.sum(hm, axis=0, keepdims=True)
        s2 = jnp.sum(hm * part, axis=0, keepdims=True)
        blk = jnp.concatenate(
            [s1, s2, jnp.zeros((6, dout), jnp.float32)], axis=0)

        @pl.when(i == 0)
        def _():
            st_ref[...] = blk

        @pl.when(i > 0)
        def _():
            st_ref[...] += blk

    return pl.pallas_call(
        body,
        grid=(NPAD // BR,),
        in_specs=[
            pl.BlockSpec((2, BR, din), lambda i: (0, i, 0)),
            pl.BlockSpec((BR, din), lambda i: (i, 0)),
            pl.BlockSpec((BR, 1), lambda i: (i, 0)),
            pl.BlockSpec((dout, din), lambda i: (0, 0)),
            pl.BlockSpec((1, dout), lambda i: (0, 0)),
        ],
        out_specs=[
            pl.BlockSpec((BR, dout), lambda i: (i, 0)),
            pl.BlockSpec((8, dout), lambda i: (0, 0)),
        ],
        out_shape=[
            jax.ShapeDtypeStruct((NPAD, dout), jnp.float32),
            jax.ShapeDtypeStruct((8, dout), jnp.float32),
        ],
    )(S, xfull, dinv, W, b)


def _bn_leaky(h, st_ref, g_ref, be_ref):
    mean = st_ref[0:1, :] / N
    var = st_ref[1:2, :] / N - mean * mean
    y = (h - mean) * lax.rsqrt(var + EPS) * g_ref[...] + be_ref[...]
    return jnp.where(y >= 0, y, 0.1 * y)


def _pass_b(h, st, dinv, g, be, dout, nco, wco):
    def body(h_ref, st_ref, dinv_ref, g_ref, be_ref, full_ref, *o_refs):
        z = dinv_ref[...] * _bn_leaky(h_ref[...], st_ref, g_ref, be_ref)
        full_ref[...] = z
        for c in range(nco):
            o_refs[c][...] = z[:, c * wco:(c + 1) * wco]

    return pl.pallas_call(
        body,
        grid=(NPAD // BR,),
        in_specs=[
            pl.BlockSpec((BR, dout), lambda i: (i, 0)),
            pl.BlockSpec((8, dout), lambda i: (0, 0)),
            pl.BlockSpec((BR, 1), lambda i: (i, 0)),
            pl.BlockSpec((1, dout), lambda i: (0, 0)),
            pl.BlockSpec((1, dout), lambda i: (0, 0)),
        ],
        out_specs=[pl.BlockSpec((BR, dout), lambda i: (i, 0))]
        + [pl.BlockSpec((BR, wco), lambda i: (i, 0)) for _ in range(nco)],
        out_shape=[jax.ShapeDtypeStruct((NPAD, dout), jnp.float32)]
        + [jax.ShapeDtypeStruct((NPAD, wco), jnp.float32)
           for _ in range(nco)],
    )(h, st, dinv, g, be)


def _pass_b5(h, st, dinv, g, be, W6):
    def body(h_ref, st_ref, dinv_ref, g_ref, be_ref, w6_ref, o_ref):
        y = _bn_leaky(h_ref[...], st_ref, g_ref, be_ref)
        t = lax.dot_general(y, w6_ref[...], (((1,), (1,)), ((), ())))
        t8 = jnp.pad(t, ((0, 0), (0, 7)))
        o_ref[...] = dinv_ref[...] * t8

    return pl.pallas_call(
        body,
        grid=(NPAD // BR,),
        in_specs=[
            pl.BlockSpec((BR, 512), lambda i: (i, 0)),
            pl.BlockSpec((8, 512), lambda i: (0, 0)),
            pl.BlockSpec((BR, 1), lambda i: (i, 0)),
            pl.BlockSpec((1, 512), lambda i: (0, 0)),
            pl.BlockSpec((1, 512), lambda i: (0, 0)),
            pl.BlockSpec((1, 512), lambda i: (0, 0)),
        ],
        out_specs=pl.BlockSpec((BR, 8), lambda i: (i, 0)),
        out_shape=jax.ShapeDtypeStruct((NPAD, 8), jnp.float32),
    )(h, st, dinv, g, be, W6)


def _prologue(x8, degS):
    def body(x_ref, d_ref, dinv_ref, o_ref):
        i = pl.program_id(0)
        deg = d_ref[0, 0, :, 0:1] + d_ref[1, 0, :, 0:1] + 1.0
        dinv = jnp.where(_row_mask(i), lax.rsqrt(deg), 0.0)
        dinv_ref[...] = dinv
        o_ref[...] = dinv * x_ref[...]

    return pl.pallas_call(
        body,
        grid=(NPAD // BR,),
        in_specs=[
            pl.BlockSpec((BR, 8), lambda i: (i, 0)),
            pl.BlockSpec((2, 1, BR, 8), lambda i: (0, 0, i, 0)),
        ],
        out_specs=[
            pl.BlockSpec((BR, 1), lambda i: (i, 0)),
            pl.BlockSpec((BR, 8), lambda i: (i, 0)),
        ],
        out_shape=[
            jax.ShapeDtypeStruct((NPAD, 1), jnp.float32),
            jax.ShapeDtypeStruct((NPAD, 8), jnp.float32),
        ],
    )(x8, degS)


def _final(S6, h6c, dinv, b6):
    def body(s_ref, h_ref, dinv_ref, b_ref, o_ref):
        v = dinv_ref[...] * (s_ref[0, 0, :, 0:1] + s_ref[1, 0, :, 0:1]
                             + h_ref[:, 0:1]) + b_ref[...]
        o_ref[...] = jax.nn.sigmoid(v)

    return pl.pallas_call(
        body,
        grid=(NPAD // BR,),
        in_specs=[
            pl.BlockSpec((2, 1, BR, 8), lambda i: (0, 0, i, 0)),
            pl.BlockSpec((BR, 8), lambda i: (i, 0)),
            pl.BlockSpec((BR, 1), lambda i: (i, 0)),
            pl.BlockSpec((1, 1), lambda i: (0, 0)),
        ],
        out_specs=pl.BlockSpec((BR, 1), lambda i: (i, 0)),
        out_shape=jax.ShapeDtypeStruct((N, 1), jnp.float32),
    )(S6, h6c, dinv, b6)


def _repack(Sflat, nc, wc):
    S = Sflat.reshape(2, nc, NPAD, wc)
    if nc == 1:
        return S.reshape(2, NPAD, wc), S
    return S.transpose(0, 2, 1, 3).reshape(2, NPAD, nc * wc), S


def kernel(x, edge_index, W1, b1, W2, b2, W3, b3, W4, b4, W5, b5, W6, b6,
           g1, be1, g2, be2, g3, be3, g4, be4, g5, be5):
    ei = edge_index.astype(jnp.int32)
    pad = jnp.full((EPAD - E,), N, jnp.int32)
    src3 = jnp.concatenate([ei[0], pad]).reshape(NW * KPT, BATCH)
    dst3 = jnp.concatenate([ei[1], pad]).reshape(NW * KPT, BATCH)

    x8 = jnp.pad(x, ((0, NPAD - N), (0, 5)))
    ones_rows = jnp.ones((BATCH, 8), jnp.float32)
    z8 = jnp.zeros((SLICE, 8), jnp.float32)
    z16 = jnp.zeros((SLICE, 16), jnp.float32)
    W1p = jnp.pad(W1, ((0, 0), (0, 5)))

    agg8 = _sc_aggregate(1, 8)
    agg4 = _sc_aggregate(4, 16)
    agg16 = _sc_aggregate(16, 16)

    degS = _sc_count()(dst3, ones_rows, z8).reshape(2, 1, NPAD, 8)
    dinv, xc0 = _prologue(x8, degS)

    S0f, _ = _repack(agg8(xc0, src3, dst3, z8), 1, 8)
    h1, st1 = _pass_a(S0f, xc0, dinv, W1p, b1[None], 8, 64)
    x1, *xc1 = _pass_b(h1, st1, dinv, g1[None], be1[None], 64, 4, 16)

    S1f, _ = _repack(agg4(*xc1, src3, dst3, z16), 4, 16)
    h2, st2 = _pass_a(S1f, x1, dinv, W2, b2[None], 64, 64)
    x2, *xc2 = _pass_b(h2, st2, dinv, g2[None], be2[None], 64, 4, 16)

    S2f, _ = _repack(agg4(*xc2, src3, dst3, z16), 4, 16)
    h3, st3 = _pass_a(S2f, x2, dinv, W3, b3[None], 64, 256)
    x3, *xc3 = _pass_b(h3, st3, dinv, g3[None], be3[None], 256, 16, 16)

    S3f, _ = _repack(agg16(*xc3, src3, dst3, z16), 16, 16)
    h4, st4 = _pass_a(S3f, x3, dinv, W4, b4[None], 256, 256)
    x4, *xc4 = _pass_b(h4, st4, dinv, g4[None], be4[None], 256, 16, 16)

    S4f, _ = _repack(agg16(*xc4, src3, dst3, z16), 16, 16)
    h5, st5 = _pass_a(S4f, x4, dinv, W5, b5[None], 256, 512)
    h6c = _pass_b5(h5, st5, dinv, g5[None], be5[None], W6)

    _, S6 = _repack(agg8(h6c, src3, dst3, z8), 1, 8)
    return _final(S6, h6c, dinv, b6[None])


# speedup vs baseline: 1.4827x; 1.0005x over previous
import functools

import jax
import jax.numpy as jnp
from jax import lax
from jax.experimental import pallas as pl
from jax.experimental.pallas import tpu as pltpu
from jax.experimental.pallas import tpu_sc as plsc

N = 50000
E = 800000
NW = 32
NS = 16
BATCH = 128
KPT = 196
EPAD = NW * KPT * BATCH
BR = 1568
NPAD = 32 * BR
SLICE = NPAD // NS
GRP = 4
EPS = 1e-5


def _sc_aggregate(nc, wc):
    mesh = plsc.VectorSubcoreMesh(core_axis_name="c", subcore_axis_name="s")
    out_t = jax.ShapeDtypeStruct((2 * nc * NPAD, wc), jnp.float32)
    scratch = [
        pltpu.VMEM((KPT, BATCH), jnp.int32),
        pltpu.VMEM((KPT, BATCH), jnp.int32),
        pltpu.VMEM((2 * GRP, BATCH, wc), jnp.float32),
        pltpu.VMEM_SHARED((NPAD, wc), jnp.float32),
        pltpu.SemaphoreType.DMA,
        pltpu.SemaphoreType.DMA,
    ]

    @functools.partial(
        pl.kernel, out_type=out_t, mesh=mesh, scratch_types=scratch,
        compiler_params=pltpu.CompilerParams(use_tc_tiling_on_sc=False))
    def k(*refs):
        x_hbm = refs[:nc]
        src_hbm, dst_hbm, zero_hbm, out_hbm = refs[nc:nc + 4]
        src_v, dst_v, rows_v, acc, gsem, ssem = refs[nc + 4:]
        c = lax.axis_index("c")
        s = lax.axis_index("s")
        wid = c * NS + s
        pltpu.sync_copy(src_hbm.at[pl.ds(wid * KPT, KPT)], src_v)
        pltpu.sync_copy(dst_hbm.at[pl.ds(wid * KPT, KPT)], dst_v)
        for cc in range(nc):
            pltpu.sync_copy(zero_hbm, acc.at[pl.ds(s * SLICE, SLICE)])
            plsc.subcore_barrier()

            for k in range(GRP):
                pltpu.async_copy(x_hbm[cc].at[src_v.at[k]],
                                 rows_v.at[k], gsem)

            def body(j, carry):
                slot = lax.rem(j, 2 * GRP)
                pltpu.make_async_copy(x_hbm[cc].at[src_v.at[j]],
                                      rows_v.at[slot], gsem).wait()

                @pl.when(j >= GRP)
                def _():
                    pltpu.make_async_copy(
                        rows_v.at[lax.rem(j + GRP, 2 * GRP)],
                        acc.at[dst_v.at[j - GRP]], ssem).wait()

                @pl.when(j < KPT - GRP)
                def _():
                    pltpu.async_copy(
                        x_hbm[cc].at[src_v.at[j + GRP]],
                        rows_v.at[lax.rem(j + GRP, 2 * GRP)], gsem)

                pltpu.async_copy(rows_v.at[slot], acc.at[dst_v.at[j]],
                                 ssem, add=True)
                return carry

            lax.fori_loop(0, KPT, body, 0)
            for k in range(GRP):
                j = KPT - GRP + k
                pltpu.make_async_copy(
                    rows_v.at[j % (2 * GRP)],
                    acc.at[dst_v.at[j]], ssem).wait()
            plsc.subcore_barrier()
            pltpu.sync_copy(
                acc.at[pl.ds(s * SLICE, SLICE)],
                out_hbm.at[pl.ds((c * nc + cc) * NPAD + s * SLICE, SLICE)])
            plsc.subcore_barrier()

    return k


def _sc_count():
    mesh = plsc.VectorSubcoreMesh(core_axis_name="c", subcore_axis_name="s")
    out_t = jax.ShapeDtypeStruct((2 * NPAD, 8), jnp.float32)
    scratch = [
        pltpu.VMEM((KPT, BATCH), jnp.int32),
        pltpu.VMEM((BATCH, 8), jnp.float32),
        pltpu.VMEM_SHARED((NPAD, 8), jnp.float32),
        pltpu.SemaphoreType.DMA,
    ]

    @functools.partial(
        pl.kernel, out_type=out_t, mesh=mesh, scratch_types=scratch,
        compiler_params=pltpu.CompilerParams(use_tc_tiling_on_sc=False))
    def k(dst_hbm, ones_hbm, zero_hbm, out_hbm, dst_v, ones_v, acc, ssem):
        c = lax.axis_index("c")
        s = lax.axis_index("s")
        wid = c * NS + s
        pltpu.sync_copy(dst_hbm.at[pl.ds(wid * KPT, KPT)], dst_v)
        pltpu.sync_copy(ones_hbm, ones_v)
        pltpu.sync_copy(zero_hbm, acc.at[pl.ds(s * SLICE, SLICE)])
        plsc.subcore_barrier()

        def fire(j, carry):
            pltpu.async_copy(ones_v, acc.at[dst_v.at[j]], ssem, add=True)

            @pl.when(j >= GRP)
            def _():
                pltpu.make_async_copy(ones_v, acc.at[dst_v.at[j - GRP]],
                                      ssem).wait()

            return carry

        lax.fori_loop(0, KPT, fire, 0)
        for k2 in range(GRP):
            pltpu.make_async_copy(ones_v, acc.at[dst_v.at[KPT - GRP + k2]],
                                  ssem).wait()
        plsc.subcore_barrier()
        pltpu.sync_copy(acc.at[pl.ds(s * SLICE, SLICE)],
                        out_hbm.at[pl.ds(c * NPAD + s * SLICE, SLICE)])
        plsc.subcore_barrier()

    return k


def _row_mask(i):
    rows = lax.broadcasted_iota(jnp.int32, (BR, 1), 0) + i * BR
    return rows < N


def _pass_a(S, xfull, dinv, W, b, din, dout):
    def body(s_ref, x_ref, dinv_ref, w_ref, b_ref, h_ref, st_ref):
        i = pl.program_id(0)
        u = dinv_ref[...] * (s_ref[0] + s_ref[1] + x_ref[...])
        part = b_ref[...] + lax.dot_general(
            u, w_ref[...], (((1,), (1,)), ((), ())))
        h_ref[...] = part
        m = _row_mask(i)
        hm = jnp.where(m, part, 0.0)
        s1 = jnp.sum(hm, axis=0, keepdims=True)
        s2 = jnp.sum(hm * part, axis=0, keepdims=True)
        blk = jnp.concatenate(
            [s1, s2, jnp.zeros((6, dout), jnp.float32)], axis=0)

        @pl.when(i == 0)
        def _():
            st_ref[...] = blk

        @pl.when(i > 0)
        def _():
            st_ref[...] += blk

    return pl.pallas_call(
        body,
        grid=(NPAD // BR,),
        in_specs=[
            pl.BlockSpec((2, BR, din), lambda i: (0, i, 0)),
            pl.BlockSpec((BR, din), lambda i: (i, 0)),
            pl.BlockSpec((BR, 1), lambda i: (i, 0)),
            pl.BlockSpec((dout, din), lambda i: (0, 0)),
            pl.BlockSpec((1, dout), lambda i: (0, 0)),
        ],
        out_specs=[
            pl.BlockSpec((BR, dout), lambda i: (i, 0)),
            pl.BlockSpec((8, dout), lambda i: (0, 0)),
        ],
        out_shape=[
            jax.ShapeDtypeStruct((NPAD, dout), jnp.float32),
            jax.ShapeDtypeStruct((8, dout), jnp.float32),
        ],
    )(S, xfull, dinv, W, b)


def _bn_leaky(h, st_ref, g_ref, be_ref):
    mean = st_ref[0:1, :] / N
    var = st_ref[1:2, :] / N - mean * mean
    y = (h - mean) * lax.rsqrt(var + EPS) * g_ref[...] + be_ref[...]
    return jnp.where(y >= 0, y, 0.1 * y)


def _pass_b(h, st, dinv, g, be, dout, nco, wco):
    def body(h_ref, st_ref, dinv_ref, g_ref, be_ref, full_ref, *o_refs):
        z = dinv_ref[...] * _bn_leaky(h_ref[...], st_ref, g_ref, be_ref)
        full_ref[...] = z
        for c in range(nco):
            o_refs[c][...] = z[:, c * wco:(c + 1) * wco]

    return pl.pallas_call(
        body,
        grid=(NPAD // BR,),
        in_specs=[
            pl.BlockSpec((BR, dout), lambda i: (i, 0)),
            pl.BlockSpec((8, dout), lambda i: (0, 0)),
            pl.BlockSpec((BR, 1), lambda i: (i, 0)),
            pl.BlockSpec((1, dout), lambda i: (0, 0)),
            pl.BlockSpec((1, dout), lambda i: (0, 0)),
        ],
        out_specs=[pl.BlockSpec((BR, dout), lambda i: (i, 0))]
        + [pl.BlockSpec((BR, wco), lambda i: (i, 0)) for _ in range(nco)],
        out_shape=[jax.ShapeDtypeStruct((NPAD, dout), jnp.float32)]
        + [jax.ShapeDtypeStruct((NPAD, wco), jnp.float32)
           for _ in range(nco)],
    )(h, st, dinv, g, be)


def _pass_b5(h, st, dinv, g, be, W6):
    def body(h_ref, st_ref, dinv_ref, g_ref, be_ref, w6_ref, o_ref):
        y = _bn_leaky(h_ref[...], st_ref, g_ref, be_ref)
        t = lax.dot_general(y, w6_ref[...], (((1,), (1,)), ((), ())))
        t8 = jnp.pad(t, ((0, 0), (0, 7)))
        o_ref[...] = dinv_ref[...] * t8

    return pl.pallas_call(
        body,
        grid=(NPAD // BR,),
        in_specs=[
            pl.BlockSpec((BR, 512), lambda i: (i, 0)),
            pl.BlockSpec((8, 512), lambda i: (0, 0)),
            pl.BlockSpec((BR, 1), lambda i: (i, 0)),
            pl.BlockSpec((1, 512), lambda i: (0, 0)),
            pl.BlockSpec((1, 512), lambda i: (0, 0)),
            pl.BlockSpec((1, 512), lambda i: (0, 0)),
        ],
        out_specs=pl.BlockSpec((BR, 8), lambda i: (i, 0)),
        out_shape=jax.ShapeDtypeStruct((NPAD, 8), jnp.float32),
    )(h, st, dinv, g, be, W6)


def _prologue(x8, degS):
    def body(x_ref, d_ref, dinv_ref, o_ref):
        i = pl.program_id(0)
        deg = d_ref[0, 0, :, 0:1] + d_ref[1, 0, :, 0:1] + 1.0
        dinv = jnp.where(_row_mask(i), lax.rsqrt(deg), 0.0)
        dinv_ref[...] = dinv
        o_ref[...] = dinv * x_ref[...]

    return pl.pallas_call(
        body,
        grid=(NPAD // BR,),
        in_specs=[
            pl.BlockSpec((BR, 8), lambda i: (i, 0)),
            pl.BlockSpec((2, 1, BR, 8), lambda i: (0, 0, i, 0)),
        ],
        out_specs=[
            pl.BlockSpec((BR, 1), lambda i: (i, 0)),
            pl.BlockSpec((BR, 8), lambda i: (i, 0)),
        ],
        out_shape=[
            jax.ShapeDtypeStruct((NPAD, 1), jnp.float32),
            jax.ShapeDtypeStruct((NPAD, 8), jnp.float32),
        ],
    )(x8, degS)


def _final(S6, h6c, dinv, b6):
    def body(s_ref, h_ref, dinv_ref, b_ref, o_ref):
        v = dinv_ref[...] * (s_ref[0, 0, :, 0:1] + s_ref[1, 0, :, 0:1]
                             + h_ref[:, 0:1]) + b_ref[...]
        o_ref[...] = jax.nn.sigmoid(v)

    return pl.pallas_call(
        body,
        grid=(NPAD // BR,),
        in_specs=[
            pl.BlockSpec((2, 1, BR, 8), lambda i: (0, 0, i, 0)),
            pl.BlockSpec((BR, 8), lambda i: (i, 0)),
            pl.BlockSpec((BR, 1), lambda i: (i, 0)),
            pl.BlockSpec((1, 1), lambda i: (0, 0)),
        ],
        out_specs=pl.BlockSpec((BR, 1), lambda i: (i, 0)),
        out_shape=jax.ShapeDtypeStruct((N, 1), jnp.float32),
    )(S6, h6c, dinv, b6)


def _repack(Sflat, nc, wc):
    S = Sflat.reshape(2, nc, NPAD, wc)
    if nc == 1:
        return S.reshape(2, NPAD, wc), S
    return S.transpose(0, 2, 1, 3).reshape(2, NPAD, nc * wc), S


def kernel(x, edge_index, W1, b1, W2, b2, W3, b3, W4, b4, W5, b5, W6, b6,
           g1, be1, g2, be2, g3, be3, g4, be4, g5, be5):
    ei = edge_index.astype(jnp.int32)
    pad = jnp.full((EPAD - E,), N, jnp.int32)
    src3 = jnp.concatenate([ei[0], pad]).reshape(NW * KPT, BATCH)
    dst3 = jnp.concatenate([ei[1], pad]).reshape(NW * KPT, BATCH)

    x8 = jnp.pad(x, ((0, NPAD - N), (0, 5)))
    ones_rows = jnp.ones((BATCH, 8), jnp.float32)
    z8 = jnp.zeros((SLICE, 8), jnp.float32)
    z16 = jnp.zeros((SLICE, 16), jnp.float32)
    W1p = jnp.pad(W1, ((0, 0), (0, 5)))

    agg8 = _sc_aggregate(1, 8)
    agg4 = _sc_aggregate(4, 16)
    agg16 = _sc_aggregate(16, 16)

    degS = _sc_count()(dst3, ones_rows, z8).reshape(2, 1, NPAD, 8)
    dinv, xc0 = _prologue(x8, degS)

    S0f, _ = _repack(agg8(xc0, src3, dst3, z8), 1, 8)
    h1, st1 = _pass_a(S0f, xc0, dinv, W1p, b1[None], 8, 64)
    x1, *xc1 = _pass_b(h1, st1, dinv, g1[None], be1[None], 64, 4, 16)

    S1f, _ = _repack(agg4(*xc1, src3, dst3, z16), 4, 16)
    h2, st2 = _pass_a(S1f, x1, dinv, W2, b2[None], 64, 64)
    x2, *xc2 = _pass_b(h2, st2, dinv, g2[None], be2[None], 64, 4, 16)

    S2f, _ = _repack(agg4(*xc2, src3, dst3, z16), 4, 16)
    h3, st3 = _pass_a(S2f, x2, dinv, W3, b3[None], 64, 256)
    x3, *xc3 = _pass_b(h3, st3, dinv, g3[None], be3[None], 256, 16, 16)

    S3f, _ = _repack(agg16(*xc3, src3, dst3, z16), 16, 16)
    h4, st4 = _pass_a(S3f, x3, dinv, W4, b4[None], 256, 256)
    x4, *xc4 = _pass_b(h4, st4, dinv, g4[None], be4[None], 256, 16, 16)

    S4f, _ = _repack(agg16(*xc4, src3, dst3, z16), 16, 16)
    h5, st5 = _pass_a(S4f, x4, dinv, W5, b5[None], 256, 512)
    h6c = _pass_b5(h5, st5, dinv, g5[None], be5[None], W6)

    _, S6 = _repack(agg8(h6c, src3, dst3, z8), 1, 8)
    return _final(S6, h6c, dinv, b6[None])


# speedup vs baseline: 1.7546x; 1.1834x over previous
import functools

import jax
import jax.numpy as jnp
from jax import lax
from jax.experimental import pallas as pl
from jax.experimental.pallas import tpu as pltpu
from jax.experimental.pallas import tpu_sc as plsc

N = 50000
E = 800000
NW = 32
NS = 16
BATCH = 128
KPT = 196
EPAD = NW * KPT * BATCH
BR = 1568
NPAD = 32 * BR
SLICE = NPAD // NS
GRP = 4
EPS = 1e-5


def _sc_aggregate(nc, wc):
    mesh = plsc.VectorSubcoreMesh(core_axis_name="c", subcore_axis_name="s")
    out_t = jax.ShapeDtypeStruct((2 * NPAD, nc * wc), jnp.float32)
    scratch = [
        pltpu.VMEM((KPT, BATCH), jnp.int32),
        pltpu.VMEM((KPT, BATCH), jnp.int32),
        pltpu.VMEM((2 * GRP, BATCH, wc), jnp.float32),
        pltpu.VMEM_SHARED((NPAD, wc), jnp.float32),
        pltpu.SemaphoreType.DMA,
        pltpu.SemaphoreType.DMA,
    ]

    @functools.partial(
        pl.kernel, out_type=out_t, mesh=mesh, scratch_types=scratch,
        compiler_params=pltpu.CompilerParams(use_tc_tiling_on_sc=False))
    def k(*refs):
        x_hbm = refs[:nc]
        src_hbm, dst_hbm, zero_hbm, out_hbm = refs[nc:nc + 4]
        src_v, dst_v, rows_v, acc, gsem, ssem = refs[nc + 4:]
        c = lax.axis_index("c")
        s = lax.axis_index("s")
        wid = c * NS + s
        pltpu.sync_copy(src_hbm.at[pl.ds(wid * KPT, KPT)], src_v)
        pltpu.sync_copy(dst_hbm.at[pl.ds(wid * KPT, KPT)], dst_v)
        for cc in range(nc):
            pltpu.sync_copy(zero_hbm, acc.at[pl.ds(s * SLICE, SLICE)])
            plsc.subcore_barrier()

            for k in range(GRP):
                pltpu.async_copy(x_hbm[cc].at[src_v.at[k]],
                                 rows_v.at[k], gsem)

            def body(j, carry):
                slot = lax.rem(j, 2 * GRP)
                pltpu.make_async_copy(x_hbm[cc].at[src_v.at[j]],
                                      rows_v.at[slot], gsem).wait()

                @pl.when(j >= GRP)
                def _():
                    pltpu.make_async_copy(
                        rows_v.at[lax.rem(j + GRP, 2 * GRP)],
                        acc.at[dst_v.at[j - GRP]], ssem).wait()

                @pl.when(j < KPT - GRP)
                def _():
                    pltpu.async_copy(
                        x_hbm[cc].at[src_v.at[j + GRP]],
                        rows_v.at[lax.rem(j + GRP, 2 * GRP)], gsem)

                pltpu.async_copy(rows_v.at[slot], acc.at[dst_v.at[j]],
                                 ssem, add=True)
                return carry

            lax.fori_loop(0, KPT, body, 0)
            for k in range(GRP):
                j = KPT - GRP + k
                pltpu.make_async_copy(
                    rows_v.at[j % (2 * GRP)],
                    acc.at[dst_v.at[j]], ssem).wait()
            plsc.subcore_barrier()
            pltpu.sync_copy(
                acc.at[pl.ds(s * SLICE, SLICE)],
                out_hbm.at[pl.ds(c * NPAD + s * SLICE, SLICE),
                           pl.ds(cc * wc, wc)])
            plsc.subcore_barrier()

    return k


def _sc_count():
    mesh = plsc.VectorSubcoreMesh(core_axis_name="c", subcore_axis_name="s")
    out_t = jax.ShapeDtypeStruct((2 * NPAD, 8), jnp.float32)
    scratch = [
        pltpu.VMEM((KPT, BATCH), jnp.int32),
        pltpu.VMEM((BATCH, 8), jnp.float32),
        pltpu.VMEM_SHARED((NPAD, 8), jnp.float32),
        pltpu.SemaphoreType.DMA,
    ]

    @functools.partial(
        pl.kernel, out_type=out_t, mesh=mesh, scratch_types=scratch,
        compiler_params=pltpu.CompilerParams(use_tc_tiling_on_sc=False))
    def k(dst_hbm, ones_hbm, zero_hbm, out_hbm, dst_v, ones_v, acc, ssem):
        c = lax.axis_index("c")
        s = lax.axis_index("s")
        wid = c * NS + s
        pltpu.sync_copy(dst_hbm.at[pl.ds(wid * KPT, KPT)], dst_v)
        pltpu.sync_copy(ones_hbm, ones_v)
        pltpu.sync_copy(zero_hbm, acc.at[pl.ds(s * SLICE, SLICE)])
        plsc.subcore_barrier()

        def fire(j, carry):
            pltpu.async_copy(ones_v, acc.at[dst_v.at[j]], ssem, add=True)

            @pl.when(j >= GRP)
            def _():
                pltpu.make_async_copy(ones_v, acc.at[dst_v.at[j - GRP]],
                                      ssem).wait()

            return carry

        lax.fori_loop(0, KPT, fire, 0)
        for k2 in range(GRP):
            pltpu.make_async_copy(ones_v, acc.at[dst_v.at[KPT - GRP + k2]],
                                  ssem).wait()
        plsc.subcore_barrier()
        pltpu.sync_copy(acc.at[pl.ds(s * SLICE, SLICE)],
                        out_hbm.at[pl.ds(c * NPAD + s * SLICE, SLICE)])
        plsc.subcore_barrier()

    return k


def _row_mask(i):
    rows = lax.broadcasted_iota(jnp.int32, (BR, 1), 0) + i * BR
    return rows < N


def _pass_a(S, xfull, dinv, W, b, din, dout):
    def body(s_ref, x_ref, dinv_ref, w_ref, b_ref, h_ref, st_ref):
        i = pl.program_id(0)
        u = dinv_ref[...] * (s_ref[0] + s_ref[1] + x_ref[...])
        part = b_ref[...] + lax.dot_general(
            u, w_ref[...], (((1,), (1,)), ((), ())))
        h_ref[...] = part
        m = _row_mask(i)
        hm = jnp.where(m, part, 0.0)
        s1 = jnp.sum(hm, axis=0, keepdims=True)
        s2 = jnp.sum(hm * part, axis=0, keepdims=True)
        blk = jnp.concatenate(
            [s1, s2, jnp.zeros((6, dout), jnp.float32)], axis=0)

        @pl.when(i == 0)
        def _():
            st_ref[...] = blk

        @pl.when(i > 0)
        def _():
            st_ref[...] += blk

    return pl.pallas_call(
        body,
        grid=(NPAD // BR,),
        in_specs=[
            pl.BlockSpec((2, BR, din), lambda i: (0, i, 0)),
            pl.BlockSpec((BR, din), lambda i: (i, 0)),
            pl.BlockSpec((BR, 1), lambda i: (i, 0)),
            pl.BlockSpec((dout, din), lambda i: (0, 0)),
            pl.BlockSpec((1, dout), lambda i: (0, 0)),
        ],
        out_specs=[
            pl.BlockSpec((BR, dout), lambda i: (i, 0)),
            pl.BlockSpec((8, dout), lambda i: (0, 0)),
        ],
        out_shape=[
            jax.ShapeDtypeStruct((NPAD, dout), jnp.float32),
            jax.ShapeDtypeStruct((8, dout), jnp.float32),
        ],
    )(S, xfull, dinv, W, b)


def _bn_leaky(h, st_ref, g_ref, be_ref):
    mean = st_ref[0:1, :] / N
    var = st_ref[1:2, :] / N - mean * mean
    y = (h - mean) * lax.rsqrt(var + EPS) * g_ref[...] + be_ref[...]
    return jnp.where(y >= 0, y, 0.1 * y)


def _pass_b(h, st, dinv, g, be, dout, nco, wco):
    def body(h_ref, st_ref, dinv_ref, g_ref, be_ref, full_ref, *o_refs):
        z = dinv_ref[...] * _bn_leaky(h_ref[...], st_ref, g_ref, be_ref)
        full_ref[...] = z
        for c in range(nco):
            o_refs[c][...] = z[:, c * wco:(c + 1) * wco]

    return pl.pallas_call(
        body,
        grid=(NPAD // BR,),
        in_specs=[
            pl.BlockSpec((BR, dout), lambda i: (i, 0)),
            pl.BlockSpec((8, dout), lambda i: (0, 0)),
            pl.BlockSpec((BR, 1), lambda i: (i, 0)),
            pl.BlockSpec((1, dout), lambda i: (0, 0)),
            pl.BlockSpec((1, dout), lambda i: (0, 0)),
        ],
        out_specs=[pl.BlockSpec((BR, dout), lambda i: (i, 0))]
        + [pl.BlockSpec((BR, wco), lambda i: (i, 0)) for _ in range(nco)],
        out_shape=[jax.ShapeDtypeStruct((NPAD, dout), jnp.float32)]
        + [jax.ShapeDtypeStruct((NPAD, wco), jnp.float32)
           for _ in range(nco)],
    )(h, st, dinv, g, be)


def _pass_b5(h, st, dinv, g, be, W6):
    def body(h_ref, st_ref, dinv_ref, g_ref, be_ref, w6_ref, o_ref):
        y = _bn_leaky(h_ref[...], st_ref, g_ref, be_ref)
        t = lax.dot_general(y, w6_ref[...], (((1,), (1,)), ((), ())))
        t8 = jnp.pad(t, ((0, 0), (0, 7)))
        o_ref[...] = dinv_ref[...] * t8

    return pl.pallas_call(
        body,
        grid=(NPAD // BR,),
        in_specs=[
            pl.BlockSpec((BR, 512), lambda i: (i, 0)),
            pl.BlockSpec((8, 512), lambda i: (0, 0)),
            pl.BlockSpec((BR, 1), lambda i: (i, 0)),
            pl.BlockSpec((1, 512), lambda i: (0, 0)),
            pl.BlockSpec((1, 512), lambda i: (0, 0)),
            pl.BlockSpec((1, 512), lambda i: (0, 0)),
        ],
        out_specs=pl.BlockSpec((BR, 8), lambda i: (i, 0)),
        out_shape=jax.ShapeDtypeStruct((NPAD, 8), jnp.float32),
    )(h, st, dinv, g, be, W6)


def _prologue(x8, degS):
    def body(x_ref, d_ref, dinv_ref, o_ref):
        i = pl.program_id(0)
        deg = d_ref[0, 0, :, 0:1] + d_ref[1, 0, :, 0:1] + 1.0
        dinv = jnp.where(_row_mask(i), lax.rsqrt(deg), 0.0)
        dinv_ref[...] = dinv
        o_ref[...] = dinv * x_ref[...]

    return pl.pallas_call(
        body,
        grid=(NPAD // BR,),
        in_specs=[
            pl.BlockSpec((BR, 8), lambda i: (i, 0)),
            pl.BlockSpec((2, 1, BR, 8), lambda i: (0, 0, i, 0)),
        ],
        out_specs=[
            pl.BlockSpec((BR, 1), lambda i: (i, 0)),
            pl.BlockSpec((BR, 8), lambda i: (i, 0)),
        ],
        out_shape=[
            jax.ShapeDtypeStruct((NPAD, 1), jnp.float32),
            jax.ShapeDtypeStruct((NPAD, 8), jnp.float32),
        ],
    )(x8, degS)


def _final(S6, h6c, dinv, b6):
    def body(s_ref, h_ref, dinv_ref, b_ref, o_ref):
        v = dinv_ref[...] * (s_ref[0, 0, :, 0:1] + s_ref[1, 0, :, 0:1]
                             + h_ref[:, 0:1]) + b_ref[...]
        o_ref[...] = jax.nn.sigmoid(v)

    return pl.pallas_call(
        body,
        grid=(NPAD // BR,),
        in_specs=[
            pl.BlockSpec((2, 1, BR, 8), lambda i: (0, 0, i, 0)),
            pl.BlockSpec((BR, 8), lambda i: (i, 0)),
            pl.BlockSpec((BR, 1), lambda i: (i, 0)),
            pl.BlockSpec((1, 1), lambda i: (0, 0)),
        ],
        out_specs=pl.BlockSpec((BR, 1), lambda i: (i, 0)),
        out_shape=jax.ShapeDtypeStruct((N, 1), jnp.float32),
    )(S6, h6c, dinv, b6)


def kernel(x, edge_index, W1, b1, W2, b2, W3, b3, W4, b4, W5, b5, W6, b6,
           g1, be1, g2, be2, g3, be3, g4, be4, g5, be5):
    ei = edge_index.astype(jnp.int32)
    pad = jnp.full((EPAD - E,), N, jnp.int32)
    src3 = jnp.concatenate([ei[0], pad]).reshape(NW * KPT, BATCH)
    dst3 = jnp.concatenate([ei[1], pad]).reshape(NW * KPT, BATCH)

    x8 = jnp.pad(x, ((0, NPAD - N), (0, 5)))
    ones_rows = jnp.ones((BATCH, 8), jnp.float32)
    z8 = jnp.zeros((SLICE, 8), jnp.float32)
    z16 = jnp.zeros((SLICE, 16), jnp.float32)
    W1p = jnp.pad(W1, ((0, 0), (0, 5)))

    agg8 = _sc_aggregate(1, 8)
    agg4 = _sc_aggregate(4, 16)
    agg16 = _sc_aggregate(16, 16)

    degS = _sc_count()(dst3, ones_rows, z8).reshape(2, 1, NPAD, 8)
    dinv, xc0 = _prologue(x8, degS)

    S0f = agg8(xc0, src3, dst3, z8).reshape(2, NPAD, 8)
    h1, st1 = _pass_a(S0f, xc0, dinv, W1p, b1[None], 8, 64)
    x1, *xc1 = _pass_b(h1, st1, dinv, g1[None], be1[None], 64, 4, 16)

    S1f = agg4(*xc1, src3, dst3, z16).reshape(2, NPAD, 64)
    h2, st2 = _pass_a(S1f, x1, dinv, W2, b2[None], 64, 64)
    x2, *xc2 = _pass_b(h2, st2, dinv, g2[None], be2[None], 64, 4, 16)

    S2f = agg4(*xc2, src3, dst3, z16).reshape(2, NPAD, 64)
    h3, st3 = _pass_a(S2f, x2, dinv, W3, b3[None], 64, 256)
    x3, *xc3 = _pass_b(h3, st3, dinv, g3[None], be3[None], 256, 16, 16)

    S3f = agg16(*xc3, src3, dst3, z16).reshape(2, NPAD, 256)
    h4, st4 = _pass_a(S3f, x3, dinv, W4, b4[None], 256, 256)
    x4, *xc4 = _pass_b(h4, st4, dinv, g4[None], be4[None], 256, 16, 16)

    S4f = agg16(*xc4, src3, dst3, z16).reshape(2, NPAD, 256)
    h5, st5 = _pass_a(S4f, x4, dinv, W5, b5[None], 256, 512)
    h6c = _pass_b5(h5, st5, dinv, g5[None], be5[None], W6)

    S6 = agg8(h6c, src3, dst3, z8).reshape(2, 1, NPAD, 8)
    return _final(S6, h6c, dinv, b6[None])
